# baseline (speedup 1.0000x reference)
import dataclasses
import functools

import jax
import jax.numpy as jnp
from jax import lax
from jax.experimental import pallas as pl
from jax.experimental.pallas import tpu as pltpu
from jax.experimental.pallas import tpu_sc as plsc

B, L, V, E, H = 1024, 50, 1000000, 64, 64
G4 = 4 * H


def _lstm_body(e_ref, wih_ref, whh_ref, b_ref, out_ref, h_ref, c_ref):
    t = pl.program_id(0)

    @pl.when(t == 0)
    def _():
        h_ref[...] = jnp.zeros((H, B), jnp.float32)
        c_ref[...] = jnp.zeros((H, B), jnp.float32)

    h = h_ref[...]
    c = c_ref[...]
    et = e_ref[...].astype(jnp.float32)

    def gate(g):
        w_i = wih_ref[pl.ds(g * H, H), :]
        w_h = whh_ref[pl.ds(g * H, H), :]
        acc = jnp.dot(w_i, et, preferred_element_type=jnp.float32)
        acc += jnp.dot(w_h, h, preferred_element_type=jnp.float32)
        return acc + b_ref[pl.ds(g * H, H), :]

    i = jax.nn.sigmoid(gate(0))
    f = jax.nn.sigmoid(gate(1))
    g = jnp.tanh(gate(2))
    o = jax.nn.sigmoid(gate(3))
    c = f * c + i * g
    h = o * jnp.tanh(c)
    h_ref[...] = h
    c_ref[...] = c
    out_ref[...] = h.reshape(1, H, B)


def _lstm_tc(e_t, wih, whh, bias2):
    return pl.pallas_call(
        _lstm_body,
        grid=(L,),
        in_specs=[
            pl.BlockSpec((E, B), lambda t: (0, t)),
            pl.BlockSpec((G4, E), lambda t: (0, 0)),
            pl.BlockSpec((G4, H), lambda t: (0, 0)),
            pl.BlockSpec((G4, 1), lambda t: (0, 0)),
        ],
        out_specs=pl.BlockSpec((1, H, B), lambda t: (t, 0, 0)),
        out_shape=jax.ShapeDtypeStruct((L, H, B), jnp.float32),
        scratch_shapes=[
            pltpu.VMEM((H, B), jnp.float32),
            pltpu.VMEM((H, B), jnp.float32),
        ],
    )(e_t, wih, whh, bias2)


def kernel(x, emb, W_ih, W_hh, b_ih, b_hh):
    eT = jnp.take(emb.astype(jnp.bfloat16).T, x.T.reshape(-1), axis=1)
    bias2 = (b_ih + b_hh).reshape(G4, 1)
    o = _lstm_tc(eT, W_ih, W_hh, bias2)
    return o.transpose(2, 0, 1)


# speedup vs baseline: 1.1296x; 1.1296x over previous
import dataclasses
import functools

import jax
import jax.numpy as jnp
from jax import lax
from jax.experimental import pallas as pl
from jax.experimental.pallas import tpu as pltpu
from jax.experimental.pallas import tpu_sc as plsc

B, L, V, E, H = 1024, 50, 1000000, 64, 64
G4 = 4 * H


def _lstm_body(e_ref, wih_ref, whh_ref, b_ref, out_ref, h_ref, c_ref):
    t = pl.program_id(0)

    @pl.when(t == 0)
    def _():
        h_ref[...] = jnp.zeros((H, B), jnp.float32)
        c_ref[...] = jnp.zeros((H, B), jnp.float32)

    h = h_ref[...].astype(jnp.bfloat16)
    c = c_ref[...]
    et = e_ref[...].astype(jnp.bfloat16)

    def gate(g):
        w_i = wih_ref[pl.ds(g * H, H), :].astype(jnp.bfloat16)
        w_h = whh_ref[pl.ds(g * H, H), :].astype(jnp.bfloat16)
        acc = jnp.dot(w_i, et, preferred_element_type=jnp.float32)
        acc += jnp.dot(w_h, h, preferred_element_type=jnp.float32)
        return acc + b_ref[pl.ds(g * H, H), :]

    def sig(z):
        return 0.5 * jnp.tanh(0.5 * z) + 0.5

    i = sig(gate(0))
    f = sig(gate(1))
    g = jnp.tanh(gate(2))
    o = sig(gate(3))
    c = f * c + i * g
    h = o * jnp.tanh(c)
    h_ref[...] = h
    c_ref[...] = c
    out_ref[...] = h.reshape(1, H, B)


def _lstm_tc(e_t, wih, whh, bias2):
    return pl.pallas_call(
        _lstm_body,
        grid=(L,),
        in_specs=[
            pl.BlockSpec((E, B), lambda t: (0, t)),
            pl.BlockSpec((G4, E), lambda t: (0, 0)),
            pl.BlockSpec((G4, H), lambda t: (0, 0)),
            pl.BlockSpec((G4, 1), lambda t: (0, 0)),
        ],
        out_specs=pl.BlockSpec((1, H, B), lambda t: (t, 0, 0)),
        out_shape=jax.ShapeDtypeStruct((L, H, B), jnp.float32),
        scratch_shapes=[
            pltpu.VMEM((H, B), jnp.float32),
            pltpu.VMEM((H, B), jnp.float32),
        ],
    )(e_t, wih, whh, bias2)


def kernel(x, emb, W_ih, W_hh, b_ih, b_hh):
    eT = jnp.take(emb.T, x.T.reshape(-1), axis=1)
    bias2 = (b_ih + b_hh).reshape(G4, 1)
    o = _lstm_tc(eT, W_ih, W_hh, bias2)
    return o.transpose(2, 0, 1)
